# baseline (speedup 1.0000x reference)
import functools

import jax
import jax.numpy as jnp
from jax import lax
from jax.experimental import pallas as pl
from jax.experimental.pallas import tpu as pltpu
from jax.experimental.pallas import tpu_sc as plsc

V = 1_000_000
VPAD = 1_048_576
NC, NS = 2, 16
NW = NC * NS
STEPS = 8
CHUNK = 12_800
NBUF = 3
SLICE = VPAD // NS
ZWORDS = 8192


def _sc_hist(x_r):
    mesh = plsc.VectorSubcoreMesh(core_axis_name="c", subcore_axis_name="s")

    @functools.partial(
        pl.kernel,
        mesh=mesh,
        out_type=jax.ShapeDtypeStruct((NC * VPAD,), jnp.float32),
        scratch_types=[
            pltpu.VMEM((CHUNK,), jnp.int32),
            pltpu.VMEM((CHUNK,), jnp.int32),
            pltpu.VMEM((CHUNK,), jnp.int32),
            pltpu.VMEM((CHUNK,), jnp.float32),
            pltpu.VMEM((ZWORDS,), jnp.float32),
            pltpu.VMEM_SHARED((VPAD,), jnp.float32),
            pltpu.SemaphoreType.DMA,
            pltpu.SemaphoreType.DMA,
            pltpu.SemaphoreType.DMA,
            pltpu.SemaphoreType.DMA,
            pltpu.SemaphoreType.DMA,
        ],
    )
    def body(x_hbm, out_hbm, idx0, idx1, idx2, ones_v, zbuf, hist,
             lsem0, lsem1, lsem2, ssem0, ssem1):
        c = lax.axis_index("c")
        s = lax.axis_index("s")
        wid = s * NC + c
        bufs = [idx0, idx1, idx2]
        lsems = [lsem0, lsem1, lsem2]
        ssems = [ssem0, ssem1]

        def off(t):
            return (wid * STEPS + t) * CHUNK

        def load(t):
            return pltpu.async_copy(
                x_hbm.at[pl.ds(off(t), CHUNK)], bufs[t % NBUF],
                lsems[t % NBUF])

        loads = {t: load(t) for t in range(min(NBUF, STEPS))}
        scats = {}

        def fill_z(i, carry):
            zbuf[pl.ds(i * 16, 16)] = jnp.zeros((16,), jnp.float32)
            return carry

        lax.fori_loop(0, ZWORDS // 16, fill_z, 0)

        def fill_o(i, carry):
            ones_v[pl.ds(i * 16, 16)] = jnp.ones((16,), jnp.float32)
            return carry

        lax.fori_loop(0, CHUNK // 16, fill_o, 0)

        base = s * SLICE
        for j in range(SLICE // ZWORDS):
            pltpu.sync_copy(zbuf, hist.at[pl.ds(base + j * ZWORDS, ZWORDS)])
        plsc.subcore_barrier()

        for t in range(STEPS):
            loads[t].wait()
            if t >= 2:
                scats[t - 2].wait()
            scats[t] = pltpu.async_copy(
                ones_v, hist.at[bufs[t % NBUF]], ssems[t % 2], add=True)
            if t + 1 >= NBUF and t + 1 < STEPS:
                loads[t + 1] = load(t + 1)
        for t in (STEPS - 2, STEPS - 1):
            scats[t].wait()
        plsc.subcore_barrier()

        pltpu.sync_copy(hist.at[pl.ds(base, SLICE)],
                        out_hbm.at[pl.ds(c * VPAD + base, SLICE)])

    return body(x_r)


def _merge(partials, counts):
    grid = 1
    rows = VPAD // 128 // grid
    blk = rows * 128

    def body(p_ref, t_ref, o_ref):
        o_ref[...] = (p_ref[0] + p_ref[1]).reshape(blk) + t_ref[...]

    return pl.pallas_call(
        body,
        grid=(grid,),
        in_specs=[
            pl.BlockSpec((NC, rows, 128), lambda i: (0, i, 0)),
            pl.BlockSpec((blk,), lambda i: (i,)),
        ],
        out_specs=pl.BlockSpec((blk,), lambda i: (i,)),
        out_shape=jax.ShapeDtypeStruct((V,), jnp.float32),
    )(partials, counts)


def kernel(x, training_embedding_counts):
    partials = _sc_hist(x.reshape(-1))
    merged = _merge(partials.reshape(NC, VPAD // 128, 128),
                    training_embedding_counts)
    return merged


# speedup vs baseline: 1.0015x; 1.0015x over previous
import functools

import jax
import jax.numpy as jnp
from jax import lax
from jax.experimental import pallas as pl
from jax.experimental.pallas import tpu as pltpu
from jax.experimental.pallas import tpu_sc as plsc

V = 1_000_000
VPAD = 1_048_576
NC, NS = 2, 16
NW = NC * NS
STEPS = 8
CHUNK = 12_800
NBUF = 3
SLICE = VPAD // NS
ZWORDS = 8192


def _sc_hist(x_r):
    mesh = plsc.VectorSubcoreMesh(core_axis_name="c", subcore_axis_name="s")

    @functools.partial(
        pl.kernel,
        mesh=mesh,
        out_type=jax.ShapeDtypeStruct((NC * VPAD,), jnp.float32),
        scratch_types=[
            pltpu.VMEM((CHUNK,), jnp.int32),
            pltpu.VMEM((CHUNK,), jnp.int32),
            pltpu.VMEM((CHUNK,), jnp.int32),
            pltpu.VMEM((CHUNK,), jnp.float32),
            pltpu.VMEM((ZWORDS,), jnp.float32),
            pltpu.VMEM_SHARED((VPAD,), jnp.float32),
            pltpu.SemaphoreType.DMA,
            pltpu.SemaphoreType.DMA,
            pltpu.SemaphoreType.DMA,
            pltpu.SemaphoreType.DMA,
            pltpu.SemaphoreType.DMA,
        ],
    )
    def body(x_hbm, out_hbm, idx0, idx1, idx2, ones_v, zbuf, hist,
             lsem0, lsem1, lsem2, ssem0, ssem1):
        c = lax.axis_index("c")
        s = lax.axis_index("s")
        wid = s * NC + c
        bufs = [idx0, idx1, idx2]
        lsems = [lsem0, lsem1, lsem2]
        ssems = [ssem0, ssem1]

        def off(t):
            return (wid * STEPS + t) * CHUNK

        def load(t):
            return pltpu.async_copy(
                x_hbm.at[pl.ds(off(t), CHUNK)], bufs[t % NBUF],
                lsems[t % NBUF])

        loads = {t: load(t) for t in range(min(NBUF, STEPS))}
        scats = {}

        def fill_z(i, carry):
            zbuf[pl.ds(i * 16, 16)] = jnp.zeros((16,), jnp.float32)
            return carry

        lax.fori_loop(0, ZWORDS // 16, fill_z, 0)

        def fill_o(i, carry):
            ones_v[pl.ds(i * 16, 16)] = jnp.ones((16,), jnp.float32)
            return carry

        lax.fori_loop(0, CHUNK // 16, fill_o, 0)

        base = s * SLICE
        for j in range(SLICE // ZWORDS):
            pltpu.sync_copy(zbuf, hist.at[pl.ds(base + j * ZWORDS, ZWORDS)])
        plsc.subcore_barrier()

        for t in range(STEPS):
            loads[t].wait()
            if t >= 2:
                scats[t - 2].wait()
            scats[t] = pltpu.async_copy(
                ones_v, hist.at[bufs[t % NBUF]], ssems[t % 2], add=True)
            if t + 1 >= NBUF and t + 1 < STEPS:
                loads[t + 1] = load(t + 1)
        for t in (STEPS - 2, STEPS - 1):
            scats[t].wait()
        plsc.subcore_barrier()

        pltpu.sync_copy(hist.at[pl.ds(base, SLICE)],
                        out_hbm.at[pl.ds(c * VPAD + base, SLICE)])

    return body(x_r)


def _merge(partials, counts):
    grid = 2
    rows = VPAD // 128 // grid
    blk = rows * 128

    def body(p_ref, t_ref, o_ref):
        o_ref[...] = (p_ref[0] + p_ref[1]).reshape(blk) + t_ref[...]

    return pl.pallas_call(
        body,
        grid=(grid,),
        in_specs=[
            pl.BlockSpec((NC, rows, 128), lambda i: (0, i, 0)),
            pl.BlockSpec((blk,), lambda i: (i,)),
        ],
        out_specs=pl.BlockSpec((blk,), lambda i: (i,)),
        out_shape=jax.ShapeDtypeStruct((V,), jnp.float32),
    )(partials, counts)


def kernel(x, training_embedding_counts):
    partials = _sc_hist(x.reshape(-1))
    merged = _merge(partials.reshape(NC, VPAD // 128, 128),
                    training_embedding_counts)
    return merged


# speedup vs baseline: 1.0254x; 1.0239x over previous
import functools

import jax
import jax.numpy as jnp
from jax import lax
from jax.experimental import pallas as pl
from jax.experimental.pallas import tpu as pltpu
from jax.experimental.pallas import tpu_sc as plsc

V = 1_000_000
VPAD = 1_048_576
NC, NS = 2, 16
NW = NC * NS
STEPS = 8
CHUNK = 12_800
NBUF = 3
SLICE = VPAD // NS
ZWORDS = 8192


def _sc_hist(x_r):
    mesh = plsc.VectorSubcoreMesh(core_axis_name="c", subcore_axis_name="s")

    @functools.partial(
        pl.kernel,
        mesh=mesh,
        out_type=jax.ShapeDtypeStruct((NC * VPAD,), jnp.float32),
        scratch_types=[
            pltpu.VMEM((CHUNK,), jnp.int32),
            pltpu.VMEM((CHUNK,), jnp.int32),
            pltpu.VMEM((CHUNK,), jnp.int32),
            pltpu.VMEM((CHUNK,), jnp.float32),
            pltpu.VMEM((ZWORDS,), jnp.float32),
            pltpu.VMEM_SHARED((VPAD,), jnp.float32),
            pltpu.SemaphoreType.DMA,
            pltpu.SemaphoreType.DMA,
            pltpu.SemaphoreType.DMA,
            pltpu.SemaphoreType.DMA,
            pltpu.SemaphoreType.DMA,
            pltpu.SemaphoreType.DMA,
        ],
    )
    def body(x_hbm, out_hbm, idx0, idx1, idx2, ones_v, zbuf, hist,
             lsem0, lsem1, lsem2, ssem0, ssem1, zsem):
        c = lax.axis_index("c")
        s = lax.axis_index("s")
        wid = s * NC + c
        bufs = [idx0, idx1, idx2]
        lsems = [lsem0, lsem1, lsem2]
        ssems = [ssem0, ssem1]

        def off(t):
            return (wid * STEPS + t) * CHUNK

        def load(t):
            return pltpu.async_copy(
                x_hbm.at[pl.ds(off(t), CHUNK)], bufs[t % NBUF],
                lsems[t % NBUF])

        loads = {t: load(t) for t in range(min(NBUF, STEPS))}
        scats = {}

        def fill_z(i, carry):
            zbuf[pl.ds(i * 16, 16)] = jnp.zeros((16,), jnp.float32)
            return carry

        lax.fori_loop(0, ZWORDS // 16, fill_z, 0)

        base = s * SLICE
        zcps = [
            pltpu.async_copy(zbuf, hist.at[pl.ds(base + j * ZWORDS, ZWORDS)],
                             zsem)
            for j in range(SLICE // ZWORDS)
        ]

        def fill_o(i, carry):
            ones_v[pl.ds(i * 16, 16)] = jnp.ones((16,), jnp.float32)
            return carry

        lax.fori_loop(0, CHUNK // 16, fill_o, 0)

        for cp in zcps:
            cp.wait()
        plsc.subcore_barrier()

        for t in range(STEPS):
            loads[t].wait()
            if t >= 2:
                scats[t - 2].wait()
            scats[t] = pltpu.async_copy(
                ones_v, hist.at[bufs[t % NBUF]], ssems[t % 2], add=True)
            if t + 1 >= NBUF and t + 1 < STEPS:
                loads[t + 1] = load(t + 1)
        for t in (STEPS - 2, STEPS - 1):
            scats[t].wait()
        plsc.subcore_barrier()

        pltpu.sync_copy(hist.at[pl.ds(base, SLICE)],
                        out_hbm.at[pl.ds(c * VPAD + base, SLICE)])

    return body(x_r)


def _merge(partials, counts):
    grid = 2
    rows = VPAD // 128 // grid
    blk = rows * 128

    def body(p_ref, t_ref, o_ref):
        o_ref[...] = (p_ref[0] + p_ref[1]).reshape(blk) + t_ref[...]

    return pl.pallas_call(
        body,
        grid=(grid,),
        in_specs=[
            pl.BlockSpec((NC, rows, 128), lambda i: (0, i, 0)),
            pl.BlockSpec((blk,), lambda i: (i,)),
        ],
        out_specs=pl.BlockSpec((blk,), lambda i: (i,)),
        out_shape=jax.ShapeDtypeStruct((V,), jnp.float32),
    )(partials, counts)


def kernel(x, training_embedding_counts):
    partials = _sc_hist(x.reshape(-1))
    merged = _merge(partials.reshape(NC, VPAD // 128, 128),
                    training_embedding_counts)
    return merged


# speedup vs baseline: 1.0262x; 1.0008x over previous
import functools

import jax
import jax.numpy as jnp
from jax import lax
from jax.experimental import pallas as pl
from jax.experimental.pallas import tpu as pltpu
from jax.experimental.pallas import tpu_sc as plsc

V = 1_000_000
VPAD = 1_048_576
NC, NS = 2, 16
NW = NC * NS
STEPS = 10
CHUNK = 10_240
NBUF = 3
SLICE = VPAD // NS
ZWORDS = 8192


def _sc_hist(x_r):
    mesh = plsc.VectorSubcoreMesh(core_axis_name="c", subcore_axis_name="s")

    @functools.partial(
        pl.kernel,
        mesh=mesh,
        out_type=jax.ShapeDtypeStruct((NC * VPAD,), jnp.float32),
        scratch_types=[
            pltpu.VMEM((CHUNK,), jnp.int32),
            pltpu.VMEM((CHUNK,), jnp.int32),
            pltpu.VMEM((CHUNK,), jnp.int32),
            pltpu.VMEM((CHUNK,), jnp.float32),
            pltpu.VMEM((ZWORDS,), jnp.float32),
            pltpu.VMEM_SHARED((VPAD,), jnp.float32),
            pltpu.SemaphoreType.DMA,
            pltpu.SemaphoreType.DMA,
            pltpu.SemaphoreType.DMA,
            pltpu.SemaphoreType.DMA,
            pltpu.SemaphoreType.DMA,
            pltpu.SemaphoreType.DMA,
        ],
    )
    def body(x_hbm, out_hbm, idx0, idx1, idx2, ones_v, zbuf, hist,
             lsem0, lsem1, lsem2, ssem0, ssem1, zsem):
        c = lax.axis_index("c")
        s = lax.axis_index("s")
        wid = s * NC + c
        bufs = [idx0, idx1, idx2]
        lsems = [lsem0, lsem1, lsem2]
        ssems = [ssem0, ssem1]

        def off(t):
            return (wid * STEPS + t) * CHUNK

        def load(t):
            return pltpu.async_copy(
                x_hbm.at[pl.ds(off(t), CHUNK)], bufs[t % NBUF],
                lsems[t % NBUF])

        loads = {t: load(t) for t in range(min(NBUF, STEPS))}
        scats = {}

        def fill_z(i, carry):
            zbuf[pl.ds(i * 16, 16)] = jnp.zeros((16,), jnp.float32)
            return carry

        lax.fori_loop(0, ZWORDS // 16, fill_z, 0)

        base = s * SLICE
        zcps = [
            pltpu.async_copy(zbuf, hist.at[pl.ds(base + j * ZWORDS, ZWORDS)],
                             zsem)
            for j in range(SLICE // ZWORDS)
        ]

        def fill_o(i, carry):
            ones_v[pl.ds(i * 16, 16)] = jnp.ones((16,), jnp.float32)
            return carry

        lax.fori_loop(0, CHUNK // 16, fill_o, 0)

        for cp in zcps:
            cp.wait()
        plsc.subcore_barrier()

        for t in range(STEPS):
            loads[t].wait()
            if t >= 2:
                scats[t - 2].wait()
            scats[t] = pltpu.async_copy(
                ones_v, hist.at[bufs[t % NBUF]], ssems[t % 2], add=True)
            if t + 1 >= NBUF and t + 1 < STEPS:
                loads[t + 1] = load(t + 1)
        for t in (STEPS - 2, STEPS - 1):
            scats[t].wait()
        plsc.subcore_barrier()

        pltpu.sync_copy(hist.at[pl.ds(base, SLICE)],
                        out_hbm.at[pl.ds(c * VPAD + base, SLICE)])

    return body(x_r)


def _merge(partials, counts):
    grid = 2
    rows = VPAD // 128 // grid
    blk = rows * 128

    def body(p_ref, t_ref, o_ref):
        o_ref[...] = (p_ref[0] + p_ref[1]).reshape(blk) + t_ref[...]

    return pl.pallas_call(
        body,
        grid=(grid,),
        in_specs=[
            pl.BlockSpec((NC, rows, 128), lambda i: (0, i, 0)),
            pl.BlockSpec((blk,), lambda i: (i,)),
        ],
        out_specs=pl.BlockSpec((blk,), lambda i: (i,)),
        out_shape=jax.ShapeDtypeStruct((V,), jnp.float32),
    )(partials, counts)


def kernel(x, training_embedding_counts):
    partials = _sc_hist(x.reshape(-1))
    merged = _merge(partials.reshape(NC, VPAD // 128, 128),
                    training_embedding_counts)
    return merged


# speedup vs baseline: 1.0271x; 1.0008x over previous
import functools

import jax
import jax.numpy as jnp
from jax import lax
from jax.experimental import pallas as pl
from jax.experimental.pallas import tpu as pltpu
from jax.experimental.pallas import tpu_sc as plsc

V = 1_000_000
VPAD = 1_048_576
NC, NS = 2, 16
NW = NC * NS
STEPS = 10
CHUNK = 10_240
NBUF = 4
SLICE = VPAD // NS
ZWORDS = 8192


def _sc_hist(x_r):
    mesh = plsc.VectorSubcoreMesh(core_axis_name="c", subcore_axis_name="s")

    @functools.partial(
        pl.kernel,
        mesh=mesh,
        out_type=jax.ShapeDtypeStruct((NC * VPAD,), jnp.float32),
        scratch_types=[
            pltpu.VMEM((CHUNK,), jnp.int32),
            pltpu.VMEM((CHUNK,), jnp.int32),
            pltpu.VMEM((CHUNK,), jnp.int32),
            pltpu.VMEM((CHUNK,), jnp.int32),
            pltpu.VMEM((CHUNK,), jnp.float32),
            pltpu.VMEM((ZWORDS,), jnp.float32),
            pltpu.VMEM_SHARED((VPAD,), jnp.float32),
            pltpu.SemaphoreType.DMA,
            pltpu.SemaphoreType.DMA,
            pltpu.SemaphoreType.DMA,
            pltpu.SemaphoreType.DMA,
            pltpu.SemaphoreType.DMA,
            pltpu.SemaphoreType.DMA,
            pltpu.SemaphoreType.DMA,
            pltpu.SemaphoreType.DMA,
        ],
    )
    def body(x_hbm, out_hbm, idx0, idx1, idx2, idx3, ones_v, zbuf, hist,
             lsem0, lsem1, lsem2, lsem3, ssem0, ssem1, ssem2, zsem):
        c = lax.axis_index("c")
        s = lax.axis_index("s")
        wid = s * NC + c
        bufs = [idx0, idx1, idx2, idx3]
        lsems = [lsem0, lsem1, lsem2, lsem3]
        ssems = [ssem0, ssem1, ssem2]

        def off(t):
            return (wid * STEPS + t) * CHUNK

        def load(t):
            return pltpu.async_copy(
                x_hbm.at[pl.ds(off(t), CHUNK)], bufs[t % NBUF],
                lsems[t % NBUF])

        loads = {t: load(t) for t in range(min(NBUF, STEPS))}
        scats = {}

        def fill_z(i, carry):
            zbuf[pl.ds(i * 16, 16)] = jnp.zeros((16,), jnp.float32)
            return carry

        lax.fori_loop(0, ZWORDS // 16, fill_z, 0)

        base = s * SLICE
        zcps = [
            pltpu.async_copy(zbuf, hist.at[pl.ds(base + j * ZWORDS, ZWORDS)],
                             zsem)
            for j in range(SLICE // ZWORDS)
        ]

        def fill_o(i, carry):
            ones_v[pl.ds(i * 16, 16)] = jnp.ones((16,), jnp.float32)
            return carry

        lax.fori_loop(0, CHUNK // 16, fill_o, 0)

        for cp in zcps:
            cp.wait()
        plsc.subcore_barrier()

        for t in range(STEPS):
            loads[t].wait()
            if t >= 3:
                scats[t - 3].wait()
            scats[t] = pltpu.async_copy(
                ones_v, hist.at[bufs[t % NBUF]], ssems[t % 3], add=True)
            if t + 1 >= NBUF and t + 1 < STEPS:
                loads[t + 1] = load(t + 1)
        for t in (STEPS - 3, STEPS - 2, STEPS - 1):
            scats[t].wait()
        plsc.subcore_barrier()

        pltpu.sync_copy(hist.at[pl.ds(base, SLICE)],
                        out_hbm.at[pl.ds(c * VPAD + base, SLICE)])

    return body(x_r)


def _merge(partials, counts):
    grid = 2
    rows = VPAD // 128 // grid
    blk = rows * 128

    def body(p_ref, t_ref, o_ref):
        o_ref[...] = (p_ref[0] + p_ref[1]).reshape(blk) + t_ref[...]

    return pl.pallas_call(
        body,
        grid=(grid,),
        in_specs=[
            pl.BlockSpec((NC, rows, 128), lambda i: (0, i, 0)),
            pl.BlockSpec((blk,), lambda i: (i,)),
        ],
        out_specs=pl.BlockSpec((blk,), lambda i: (i,)),
        out_shape=jax.ShapeDtypeStruct((V,), jnp.float32),
    )(partials, counts)


def kernel(x, training_embedding_counts):
    partials = _sc_hist(x.reshape(-1))
    merged = _merge(partials.reshape(NC, VPAD // 128, 128),
                    training_embedding_counts)
    return merged
